# baseline (speedup 1.0000x reference)
import functools

import jax
import jax.numpy as jnp
from jax import lax
from jax.experimental import pallas as pl
from jax.experimental.pallas import tpu as pltpu
from jax.experimental.pallas import tpu_sc as plsc

NSEG = 1024
NROWS = 320000
D = 128
NW = 32
UNITS = NROWS // 128
WU = 80
LAST_WU = UNITS - 31 * WU
NBUF = 4
LOOKAHEAD = 3
WIN = 128
ACC_ROWS = NSEG + WIN


def _sc_segment_sum(scaler, batch_index2d, zeros):
    mesh = plsc.VectorSubcoreMesh(core_axis_name="c", subcore_axis_name="s")

    @functools.partial(
        pl.kernel,
        mesh=mesh,
        out_type=jax.ShapeDtypeStruct((2 * NSEG, D), jnp.float32),
        scratch_types=[
            pltpu.VMEM_SHARED((ACC_ROWS, D), jnp.float32),
            pltpu.VMEM((NBUF * 128, D), jnp.float32),
            pltpu.VMEM((WIN, D), jnp.float32),
            pltpu.VMEM((WU, 128), jnp.int32),
            pltpu.VMEM((WIN,), jnp.int32),
            pltpu.VMEM((16,), jnp.int32),
            pltpu.VMEM((NSEG // 16, D), jnp.float32),
            pltpu.SMEM((1,), jnp.int32),
            pltpu.SemaphoreType.DMA,
            pltpu.SemaphoreType.DMA,
            pltpu.SemaphoreType.DMA,
            pltpu.SemaphoreType.DMA,
        ],
    )
    def seg_sum(scaler_hbm, idx_hbm, zeros_hbm, out_hbm,
                acc, rows, win, idx_all, flidx, flidx16, bounce, wbase,
                *ld_sems):
        c = lax.axis_index("c")
        s = lax.axis_index("s")
        wid = s * 2 + c

        seg0 = s * (NSEG // 16)
        pltpu.sync_copy(zeros_hbm.at[pl.ds(seg0, NSEG // 16)], bounce)
        pltpu.sync_copy(bounce, acc.at[pl.ds(seg0, NSEG // 16)])
        pltpu.sync_copy(zeros_hbm.at[pl.ds(0, WIN)], win)
        plsc.subcore_barrier()

        start_u = wid * WU
        nu = jnp.where(wid < NW - 1, WU, LAST_WU)

        pltpu.sync_copy(idx_hbm.at[pl.ds(start_u, WU)], idx_all)
        wbase[0] = idx_all[0, pl.ds(0, 16)][0]

        def buf(b):
            return rows.at[pl.ds(b * 128, 128)]

        def load(u, b):
            base = (start_u + u) * 128
            pltpu.make_async_copy(
                scaler_hbm.at[pl.ds(base, 128)], buf(b), ld_sems[b]).start()

        def drain_ld(b):
            pltpu.make_async_copy(
                scaler_hbm.at[pl.ds(0, 128)], buf(b), ld_sems[b]).wait()

        def flush_window():
            wb = wbase[0]
            for kk in range(WIN // 16):
                flidx[pl.ds(kk * 16, 16)] = (
                    wb + kk * 16 + lax.iota(jnp.int32, 16))
            pltpu.sync_copy(win, acc.at[flidx], add=True)
            pltpu.sync_copy(zeros_hbm.at[pl.ds(0, WIN)], win)

        def process_unit(u, b):
            def group(g, _):
                vidx = idx_all[u, pl.ds(g * 16, 16)]
                lo = vidx[0]
                hi = vidx[15]
                row0 = g * 16

                @pl.when(hi - lo >= WIN)
                def _():
                    flidx16[...] = vidx
                    pltpu.sync_copy(rows.at[pl.ds(b * 128 + row0, 16)],
                                    acc.at[flidx16], add=True)

                @pl.when(hi - lo < WIN)
                def _():
                    @pl.when(hi - wbase[0] >= WIN)
                    def _():
                        flush_window()
                        wbase[0] = lo

                    wb = wbase[0]

                    @pl.when(lo == hi)
                    def _():
                        wrow = lo - wb
                        for v in range(8):
                            sv = rows[b * 128 + row0, pl.ds(v * 16, 16)]
                            for lane in range(1, 16):
                                sv = sv + rows[b * 128 + row0 + lane,
                                               pl.ds(v * 16, 16)]
                            plsc.addupdate(win.at[wrow, pl.ds(v * 16, 16)], sv)

                    @pl.when(lo != hi)
                    def _():
                        for lane in range(16):
                            wrow = vidx[lane] - wb
                            for v in range(8):
                                plsc.addupdate(
                                    win.at[wrow, pl.ds(v * 16, 16)],
                                    rows[b * 128 + row0 + lane,
                                         pl.ds(v * 16, 16)])
                return ()

            lax.fori_loop(0, 128 // 16, group, ())

        for p in range(LOOKAHEAD):
            load(p, p)

        def body(i, _):
            for k in range(NBUF):
                u = i * NBUF + k
                drain_ld(k)

                @pl.when(u + LOOKAHEAD < nu)
                def _():
                    load(u + LOOKAHEAD, (k + LOOKAHEAD) % NBUF)

                process_unit(u, k)
            return ()

        lax.fori_loop(0, nu // NBUF, body, ())
        flush_window()
        plsc.subcore_barrier()

        pltpu.sync_copy(acc.at[pl.ds(seg0, NSEG // 16)], bounce)
        pltpu.sync_copy(bounce, out_hbm.at[pl.ds(c * NSEG + seg0, NSEG // 16)])

    return seg_sum(scaler, batch_index2d, zeros)


def _mlp_body(p_ref, w1_ref, b1_ref, w2_ref, b2_ref, o_ref):
    x = p_ref[0:NSEG, :] + p_ref[NSEG:2 * NSEG, :]
    h = jnp.dot(x, w1_ref[...], preferred_element_type=jnp.float32) + b1_ref[...]
    h = jnp.maximum(h, 0.0)
    o = jnp.sum(h * w2_ref[...], axis=1, keepdims=True) + b2_ref[0, 0]
    o_ref[...] = o


def _mlp(partials, W1, b1, W2, b2):
    return pl.pallas_call(
        _mlp_body,
        out_shape=jax.ShapeDtypeStruct((NSEG, 1), jnp.float32),
    )(partials, W1, b1.reshape(1, D), W2.reshape(1, D), b2.reshape(1, 1))


def kernel(scaler, vector, batch_index, W1, b1, W2, b2):
    zeros = jnp.zeros((NSEG, D), jnp.float32)
    idx2d = jnp.pad(batch_index.reshape(UNITS, 128), ((0, NW * WU - UNITS), (0, 0)))
    partials = _sc_segment_sum(scaler, idx2d, zeros)
    return _mlp(partials, W1, b1, W2, b2)


# speedup vs baseline: 1.6499x; 1.6499x over previous
import functools

import jax
import jax.numpy as jnp
from jax import lax
from jax.experimental import pallas as pl
from jax.experimental.pallas import tpu as pltpu
from jax.experimental.pallas import tpu_sc as plsc

NSEG = 1024
NROWS = 320000
D = 128
NW = 32
UNITS = NROWS // 128
WU = 80
LAST_WU = UNITS - 31 * WU
NBUF = 5
LOOKAHEAD = 2
SC_LAG = 3


def _sc_segment_sum(scaler, batch_index2d, zeros):
    mesh = plsc.VectorSubcoreMesh(core_axis_name="c", subcore_axis_name="s")

    @functools.partial(
        pl.kernel,
        mesh=mesh,
        out_type=jax.ShapeDtypeStruct((2 * NSEG, D), jnp.float32),
        scratch_types=[
            pltpu.VMEM_SHARED((NSEG, D), jnp.float32),
            pltpu.VMEM((NBUF * 128, D), jnp.float32),
            pltpu.VMEM((WU, 128), jnp.int32),
            pltpu.VMEM((NSEG // 16, D), jnp.float32),
            pltpu.SemaphoreType.DMA,
            pltpu.SemaphoreType.DMA,
            pltpu.SemaphoreType.DMA,
            pltpu.SemaphoreType.DMA,
            pltpu.SemaphoreType.DMA,
            pltpu.SemaphoreType.DMA,
            pltpu.SemaphoreType.DMA,
            pltpu.SemaphoreType.DMA,
            pltpu.SemaphoreType.DMA,
            pltpu.SemaphoreType.DMA,
        ],
    )
    def seg_sum(scaler_hbm, idx_hbm, zeros_hbm, out_hbm,
                acc, rows, idx_all, bounce, *sems):
        ld_sems = sems[:NBUF]
        sc_sems = sems[NBUF:]
        c = lax.axis_index("c")
        s = lax.axis_index("s")
        wid = s * 2 + c

        seg0 = s * (NSEG // 16)
        pltpu.sync_copy(zeros_hbm.at[pl.ds(seg0, NSEG // 16)], bounce)
        pltpu.sync_copy(bounce, acc.at[pl.ds(seg0, NSEG // 16)])
        plsc.subcore_barrier()

        start_u = wid * WU
        nchunk = jnp.where(wid < NW - 1, WU, LAST_WU)

        pltpu.sync_copy(idx_hbm.at[pl.ds(start_u, WU)], idx_all)

        def buf(b):
            return rows.at[pl.ds(b * 128, 128)]

        def load(chunk, b):
            base = (start_u + chunk) * 128
            pltpu.make_async_copy(
                scaler_hbm.at[pl.ds(base, 128)], buf(b), ld_sems[b]).start()

        def drain_ld(b):
            pltpu.make_async_copy(
                scaler_hbm.at[pl.ds(0, 128)], buf(b), ld_sems[b]).wait()

        def scatter(chunk, b):
            pltpu.make_async_copy(
                buf(b), acc.at[idx_all.at[chunk]], sc_sems[b]).start(add=True)

        def drain_sc(b):
            pltpu.make_async_copy(
                buf(b), acc.at[idx_all.at[0]], sc_sems[b]).wait()

        for p in range(LOOKAHEAD):
            load(p, p)

        def body(i, _):
            for k in range(NBUF):
                chunk = i * NBUF + k

                @pl.when(chunk >= SC_LAG)
                def _():
                    drain_sc((k - SC_LAG) % NBUF)

                drain_ld(k)
                scatter(chunk, k)

                @pl.when(chunk + LOOKAHEAD < nchunk)
                def _():
                    load(chunk + LOOKAHEAD, (k + LOOKAHEAD) % NBUF)
            return ()

        lax.fori_loop(0, nchunk // NBUF, body, ())
        for t in range(SC_LAG):
            drain_sc((NBUF - SC_LAG + t) % NBUF)
        plsc.subcore_barrier()

        pltpu.sync_copy(acc.at[pl.ds(seg0, NSEG // 16)], bounce)
        pltpu.sync_copy(bounce, out_hbm.at[pl.ds(c * NSEG + seg0, NSEG // 16)])

    return seg_sum(scaler, batch_index2d, zeros)


def _mlp_body(p_ref, w1_ref, b1_ref, w2_ref, b2_ref, o_ref):
    x = p_ref[0:NSEG, :] + p_ref[NSEG:2 * NSEG, :]
    h = jnp.dot(x, w1_ref[...], preferred_element_type=jnp.float32) + b1_ref[...]
    h = jnp.maximum(h, 0.0)
    o = jnp.sum(h * w2_ref[...], axis=1, keepdims=True) + b2_ref[0, 0]
    o_ref[...] = o


def _mlp(partials, W1, b1, W2, b2):
    return pl.pallas_call(
        _mlp_body,
        out_shape=jax.ShapeDtypeStruct((NSEG, 1), jnp.float32),
    )(partials, W1, b1.reshape(1, D), W2.reshape(1, D), b2.reshape(1, 1))


def kernel(scaler, vector, batch_index, W1, b1, W2, b2):
    zeros = jnp.zeros((NSEG, D), jnp.float32)
    idx2d = jnp.pad(batch_index.reshape(UNITS, 128), ((0, NW * WU - UNITS), (0, 0)))
    partials = _sc_segment_sum(scaler, idx2d, zeros)
    return _mlp(partials, W1, b1, W2, b2)


# speedup vs baseline: 1.8459x; 1.1188x over previous
import functools

import jax
import jax.numpy as jnp
from jax import lax
from jax.experimental import pallas as pl
from jax.experimental.pallas import tpu as pltpu
from jax.experimental.pallas import tpu_sc as plsc

NSEG = 1024
NROWS = 320000
D = 128
NW = 32
UNITS = NROWS // 128
WU = 80
LAST_WU = UNITS - 31 * WU
NBUF = 5
LOOKAHEAD = 4
SC_LAG = 1


def _sc_segment_sum(scaler, batch_index2d, zeros):
    mesh = plsc.VectorSubcoreMesh(core_axis_name="c", subcore_axis_name="s")

    @functools.partial(
        pl.kernel,
        mesh=mesh,
        out_type=jax.ShapeDtypeStruct((2 * NSEG, D), jnp.float32),
        scratch_types=[
            pltpu.VMEM_SHARED((NSEG, D), jnp.float32),
            pltpu.VMEM((NBUF * 128, D), jnp.float32),
            pltpu.VMEM((WU, 128), jnp.int32),
            pltpu.VMEM((NSEG // 16, D), jnp.float32),
            pltpu.SemaphoreType.DMA,
            pltpu.SemaphoreType.DMA,
            pltpu.SemaphoreType.DMA,
            pltpu.SemaphoreType.DMA,
            pltpu.SemaphoreType.DMA,
            pltpu.SemaphoreType.DMA,
            pltpu.SemaphoreType.DMA,
            pltpu.SemaphoreType.DMA,
            pltpu.SemaphoreType.DMA,
            pltpu.SemaphoreType.DMA,
        ],
    )
    def seg_sum(scaler_hbm, idx_hbm, zeros_hbm, out_hbm,
                acc, rows, idx_all, bounce, *sems):
        ld_sems = sems[:NBUF]
        sc_sems = sems[NBUF:]
        c = lax.axis_index("c")
        s = lax.axis_index("s")
        wid = s * 2 + c

        seg0 = s * (NSEG // 16)
        pltpu.sync_copy(zeros_hbm.at[pl.ds(seg0, NSEG // 16)], bounce)
        pltpu.sync_copy(bounce, acc.at[pl.ds(seg0, NSEG // 16)])
        plsc.subcore_barrier()

        start_u = wid * WU
        nchunk = jnp.where(wid < NW - 1, WU, LAST_WU)

        pltpu.sync_copy(idx_hbm.at[pl.ds(start_u, WU)], idx_all)

        def buf(b):
            return rows.at[pl.ds(b * 128, 128)]

        def load(chunk, b):
            base = (start_u + chunk) * 128
            pltpu.make_async_copy(
                scaler_hbm.at[pl.ds(base, 128)], buf(b), ld_sems[b]).start()

        def drain_ld(b):
            pltpu.make_async_copy(
                scaler_hbm.at[pl.ds(0, 128)], buf(b), ld_sems[b]).wait()

        def scatter(chunk, b):
            pltpu.make_async_copy(
                buf(b), acc.at[idx_all.at[chunk]], sc_sems[b]).start(add=True)

        def drain_sc(b):
            pltpu.make_async_copy(
                buf(b), acc.at[idx_all.at[0]], sc_sems[b]).wait()

        for p in range(LOOKAHEAD):
            load(p, p)

        def body(i, _):
            for k in range(NBUF):
                chunk = i * NBUF + k

                @pl.when(chunk >= SC_LAG)
                def _():
                    drain_sc((k - SC_LAG) % NBUF)

                drain_ld(k)
                scatter(chunk, k)

                @pl.when(chunk + LOOKAHEAD < nchunk)
                def _():
                    load(chunk + LOOKAHEAD, (k + LOOKAHEAD) % NBUF)
            return ()

        lax.fori_loop(0, nchunk // NBUF, body, ())
        for t in range(SC_LAG):
            drain_sc((NBUF - SC_LAG + t) % NBUF)
        plsc.subcore_barrier()

        pltpu.sync_copy(acc.at[pl.ds(seg0, NSEG // 16)], bounce)
        pltpu.sync_copy(bounce, out_hbm.at[pl.ds(c * NSEG + seg0, NSEG // 16)])

    return seg_sum(scaler, batch_index2d, zeros)


def _mlp_body(p_ref, w1_ref, b1_ref, w2_ref, b2_ref, o_ref):
    x = p_ref[0:NSEG, :] + p_ref[NSEG:2 * NSEG, :]
    h = jnp.dot(x, w1_ref[...], preferred_element_type=jnp.float32) + b1_ref[...]
    h = jnp.maximum(h, 0.0)
    o = jnp.sum(h * w2_ref[...], axis=1, keepdims=True) + b2_ref[0, 0]
    o_ref[...] = o


def _mlp(partials, W1, b1, W2, b2):
    return pl.pallas_call(
        _mlp_body,
        out_shape=jax.ShapeDtypeStruct((NSEG, 1), jnp.float32),
    )(partials, W1, b1.reshape(1, D), W2.reshape(1, D), b2.reshape(1, 1))


def kernel(scaler, vector, batch_index, W1, b1, W2, b2):
    zeros = jnp.zeros((NSEG, D), jnp.float32)
    idx2d = jnp.pad(batch_index.reshape(UNITS, 128), ((0, NW * WU - UNITS), (0, 0)))
    partials = _sc_segment_sum(scaler, idx2d, zeros)
    return _mlp(partials, W1, b1, W2, b2)
